# baseline (speedup 1.0000x reference)
import functools

import jax
import jax.numpy as jnp
from jax import lax
from jax.experimental import pallas as pl
from jax.experimental.pallas import tpu as pltpu
from jax.experimental.pallas import tpu_sc as plsc

_N = 10000
_E = 320000
_D = 128
_G = 16

_HALF = 5056
_JUNK = 32
_ACC = _HALF + _JUNK
_NP = 2 * _HALF

_CH = 125
_NCH = 160
_RPT = 320
_WCH = 128


def _make_edge_agg():
    mesh = plsc.VectorSubcoreMesh(core_axis_name="c", subcore_axis_name="s")

    @functools.partial(
        pl.kernel,
        mesh=mesh,
        out_type=jax.ShapeDtypeStruct((_NP, _D), jnp.float32),
        scratch_types=[
            pltpu.VMEM((_NCH, _CH), jnp.int32),
            pltpu.VMEM((_NCH, _CH), jnp.int32),
            pltpu.VMEM((2, _CH, _D), jnp.float32),
            pltpu.VMEM((_WCH, _D), jnp.float32),
            pltpu.VMEM_SHARED((_ACC, _D), jnp.float32),
            pltpu.SemaphoreType.DMA,
            pltpu.SemaphoreType.DMA,
        ],
    )
    def agg(src_hbm, dstp_hbm, x_hbm, out_hbm, src_v, dst_v, rows_v, stage_v,
            acc_sh, sem0, sem1):
        c = lax.axis_index("c")
        s = lax.axis_index("s")
        sems = (sem0, sem1)

        def _zrow(j, carry):
            for k in range(_D // 16):
                stage_v[j, pl.ds(k * 16, 16)] = jnp.zeros((16,), jnp.float32)
            return carry

        lax.fori_loop(0, _WCH, _zrow, 0)
        for k2 in range(2):
            pltpu.sync_copy(stage_v,
                            acc_sh.at[pl.ds(s * _RPT + k2 * _WCH, _WCH)])

        @pl.when(s < 15)
        def _():
            pltpu.sync_copy(stage_v.at[pl.ds(0, 64)],
                            acc_sh.at[pl.ds(s * _RPT + 2 * _WCH, 64)])

        pltpu.sync_copy(src_hbm.at[s], src_v)
        pltpu.sync_copy(dstp_hbm.at[c, s], dst_v)
        plsc.subcore_barrier()

        def _gather(j, b):
            return pltpu.make_async_copy(x_hbm.at[src_v.at[j]],
                                         rows_v.at[b], sems[b])

        _gather(0, 0).start()

        def _body(t, carry):
            for b in range(2):
                j = t * 2 + b
                _gather(j, b).wait()

                @pl.when(j + 1 < _NCH)
                def _():
                    _gather(j + 1, 1 - b).start()

                pltpu.sync_copy(rows_v.at[b], acc_sh.at[dst_v.at[j]],
                                add=True)
            return carry

        lax.fori_loop(0, _NCH // 2, _body, 0)
        plsc.subcore_barrier()

        for k2 in range(2):
            base = s * _RPT + k2 * _WCH
            pltpu.sync_copy(acc_sh.at[pl.ds(base, _WCH)], stage_v)
            pltpu.sync_copy(stage_v, out_hbm.at[pl.ds(c * _HALF + base,
                                                      _WCH)])

        @pl.when(s < 15)
        def _():
            base = s * _RPT + 2 * _WCH
            pltpu.sync_copy(acc_sh.at[pl.ds(base, 64)],
                            stage_v.at[pl.ds(0, 64)])
            pltpu.sync_copy(stage_v.at[pl.ds(0, 64)],
                            out_hbm.at[pl.ds(c * _HALF + base, 64)])

    return agg


_edge_agg = _make_edge_agg()

_BLK = 1000
_EROWS = _E // _D


def _remap_body(d_ref, o_ref):
    d = d_ref[...]
    col = lax.broadcasted_iota(jnp.int32, (_EROWS, _D), 1)
    row = lax.broadcasted_iota(jnp.int32, (_EROWS, _D), 0)
    jnk = _HALF + lax.rem(col + row * 37, _JUNK)
    for c in range(2):
        local = d - c * _HALF
        ok = (local >= 0) & (local < _HALF)
        o_ref[c] = jnp.where(ok, local, jnk)


def _tc_remap(dst2):
    return pl.pallas_call(
        _remap_body,
        out_shape=jax.ShapeDtypeStruct((2, _EROWS, _D), jnp.int32),
    )(dst2)


def _layer_body(eps_ref, x_ref, agg_ref, bat_ref, w_ref, b_ref,
                o_ref, po_ref, co_ref, pooled, cnt):
    i = pl.program_id(0)

    @pl.when(i == 0)
    def _():
        pooled[...] = jnp.zeros_like(pooled)
        cnt[...] = jnp.zeros_like(cnt)

    sums = eps_ref[0, 0] * x_ref[...] + agg_ref[...]
    h = jnp.dot(sums, w_ref[...], preferred_element_type=jnp.float32)
    o_ref[...] = jnp.maximum(h + b_ref[...], 0.0)

    bat = bat_ref[0, 0, :]
    mask = (bat[:, None] == lax.broadcasted_iota(jnp.int32, (_BLK, _G), 1)
            ).astype(jnp.float32)
    dims = (((0,), (0,)), ((), ()))
    pooled[...] += lax.dot_general(mask, sums, dims,
                                   preferred_element_type=jnp.float32)
    cnt[...] += lax.dot_general(mask, jnp.ones((_BLK, _D), jnp.float32), dims,
                                preferred_element_type=jnp.float32)

    @pl.when(i == pl.num_programs(0) - 1)
    def _():
        po_ref[...] = pooled[...]
        co_ref[...] = cnt[...]


def _tc_layer(epsp, x, agg, batch3, w, b):
    return pl.pallas_call(
        _layer_body,
        grid=(_N // _BLK,),
        in_specs=[
            pl.BlockSpec((1, 1), lambda i: (0, 0)),
            pl.BlockSpec((_BLK, _D), lambda i: (i, 0)),
            pl.BlockSpec((_BLK, _D), lambda i: (i, 0)),
            pl.BlockSpec((1, 1, _BLK), lambda i: (i, 0, 0)),
            pl.BlockSpec((_D, _D), lambda i: (0, 0)),
            pl.BlockSpec((1, _D), lambda i: (0, 0)),
        ],
        out_specs=[
            pl.BlockSpec((_BLK, _D), lambda i: (i, 0)),
            pl.BlockSpec((_G, _D), lambda i: (0, 0)),
            pl.BlockSpec((_G, _D), lambda i: (0, 0)),
        ],
        out_shape=[
            jax.ShapeDtypeStruct((_N, _D), jnp.float32),
            jax.ShapeDtypeStruct((_G, _D), jnp.float32),
            jax.ShapeDtypeStruct((_G, _D), jnp.float32),
        ],
        scratch_shapes=[
            pltpu.VMEM((_G, _D), jnp.float32),
            pltpu.VMEM((_G, _D), jnp.float32),
        ],
    )(epsp, x, agg, batch3, w, b)


def _final_body(p_ref, c_ref, w_ref, b_ref, o_ref):
    o_ref[...] = (jnp.dot(p_ref[...], w_ref[...],
                          preferred_element_type=jnp.float32)
                  + c_ref[...] * b_ref[...])


def _tc_final(pooled, cnt, w, b):
    return pl.pallas_call(
        _final_body,
        out_shape=jax.ShapeDtypeStruct((_G, _D), jnp.float32),
    )(pooled, cnt, w, b)


def kernel(x, edge_index, batch, eps0, W0, b0, eps1, W1, b1, eps2, W2, b2):
    src = edge_index[0].reshape(16, _NCH, _CH)
    dstp = _tc_remap(edge_index[1].reshape(_E // _D, _D))
    dstp = dstp.reshape(2, 16, _NCH, _CH)
    batch3 = batch.reshape(_N // _BLK, 1, _BLK)
    epss = jnp.stack([(1.0 + eps0).reshape(1, 1),
                      (1.0 + eps1).reshape(1, 1),
                      (1.0 + eps2).reshape(1, 1)])
    ws = jnp.stack([W0, W1, W2])
    bs = jnp.stack([b0.reshape(1, _D), b1.reshape(1, _D),
                    b2.reshape(1, _D)])

    def _scan_body(h, lw):
        eps_l, w_l, b_l = lw
        agg = _edge_agg(src, dstp, h)
        h2, pooled, cnt = _tc_layer(eps_l, h, agg, batch3, w_l, b_l)
        return h2, (pooled, cnt)

    _, (pooleds, cnts) = lax.scan(_scan_body, x, (epss, ws, bs))
    return _tc_final(pooleds[-1], cnts[-1], W2, b2.reshape(1, _D))


# speedup vs baseline: 1.6061x; 1.6061x over previous
import functools

import jax
import jax.numpy as jnp
from jax import lax
from jax.experimental import pallas as pl
from jax.experimental.pallas import tpu as pltpu
from jax.experimental.pallas import tpu_sc as plsc

_N = 10000
_E = 320000
_D = 128
_G = 16

_ACC = 10112
_CH = 125
_NCH = 80
_RING = 16
_RPT = _ACC // 16
_WCH = 64


def _make_edge_agg():
    mesh = plsc.VectorSubcoreMesh(core_axis_name="c", subcore_axis_name="s")

    @functools.partial(
        pl.kernel,
        mesh=mesh,
        out_type=jax.ShapeDtypeStruct((2, _ACC, _D), jnp.float32),
        scratch_types=[
            pltpu.VMEM((_RING, _CH), jnp.int32),
            pltpu.VMEM((_RING, _CH), jnp.int32),
            pltpu.VMEM((2, _CH, _D), jnp.float32),
            pltpu.VMEM((_WCH, _D), jnp.float32),
            pltpu.VMEM_SHARED((_ACC, _D), jnp.float32),
            pltpu.SemaphoreType.DMA,
            pltpu.SemaphoreType.DMA,
        ],
    )
    def agg(src_hbm, dst_hbm, x_hbm, out_hbm, src_v, dst_v, rows_v, stage_v,
            acc_sh, sem0, sem1):
        c = lax.axis_index("c")
        s = lax.axis_index("s")
        wid = s * 2 + c
        sems = (sem0, sem1)

        def _zrow(j, carry):
            for k in range(_D // 16):
                stage_v[j, pl.ds(k * 16, 16)] = jnp.zeros((16,), jnp.float32)
            return carry

        lax.fori_loop(0, _WCH, _zrow, 0)
        for k2 in range(9):
            pltpu.sync_copy(stage_v,
                            acc_sh.at[pl.ds(s * _RPT + k2 * _WCH, _WCH)])
        pltpu.sync_copy(stage_v.at[pl.ds(0, 56)],
                        acc_sh.at[pl.ds(s * _RPT + 9 * _WCH, 56)])

        def _refill(blk):
            pltpu.sync_copy(src_hbm.at[wid, pl.ds(blk * _RING, _RING)],
                            src_v)
            pltpu.sync_copy(dst_hbm.at[wid, pl.ds(blk * _RING, _RING)],
                            dst_v)

        _refill(0)
        plsc.subcore_barrier()

        def _gather(r, b):
            return pltpu.make_async_copy(x_hbm.at[src_v.at[r]],
                                         rows_v.at[b], sems[b])

        _gather(0, 0).start()

        def _body(t, carry):
            for b in range(2):
                j = t * 2 + b
                r = lax.rem(j, _RING)
                _gather(r, b).wait()

                @pl.when((j + 1 < _NCH) & (r != _RING - 1))
                def _():
                    _gather(r + 1, 1 - b).start()

                pltpu.sync_copy(rows_v.at[b], acc_sh.at[dst_v.at[r]],
                                add=True)

                @pl.when((j + 1 < _NCH) & (r == _RING - 1))
                def _():
                    _refill((j + 1) // _RING)
                    _gather(0, 1 - b).start()
            return carry

        lax.fori_loop(0, _NCH // 2, _body, 0)
        plsc.subcore_barrier()

        for k2 in range(9):
            base = s * _RPT + k2 * _WCH
            pltpu.sync_copy(acc_sh.at[pl.ds(base, _WCH)], stage_v)
            pltpu.sync_copy(stage_v, out_hbm.at[c, pl.ds(base, _WCH)])
        base = s * _RPT + 9 * _WCH
        pltpu.sync_copy(acc_sh.at[pl.ds(base, 56)],
                        stage_v.at[pl.ds(0, 56)])
        pltpu.sync_copy(stage_v.at[pl.ds(0, 56)],
                        out_hbm.at[c, pl.ds(base, 56)])

    return agg


_edge_agg = _make_edge_agg()

_BLK = 1000


def _layer_body(eps_ref, x_ref, agg_ref, bat_ref, w_ref, b_ref,
                o_ref, po_ref, co_ref, pooled, cnt):
    i = pl.program_id(0)

    @pl.when(i == 0)
    def _():
        pooled[...] = jnp.zeros_like(pooled)
        cnt[...] = jnp.zeros_like(cnt)

    sums = eps_ref[0, 0] * x_ref[...] + agg_ref[0] + agg_ref[1]
    h = jnp.dot(sums, w_ref[...], preferred_element_type=jnp.float32)
    o_ref[...] = jnp.maximum(h + b_ref[...], 0.0)

    bat = bat_ref[0, 0, :]
    mask = (bat[:, None] == lax.broadcasted_iota(jnp.int32, (_BLK, _G), 1)
            ).astype(jnp.float32)
    dims = (((0,), (0,)), ((), ()))
    pooled[...] += lax.dot_general(mask, sums, dims,
                                   preferred_element_type=jnp.float32)
    cnt[...] += lax.dot_general(mask, jnp.ones((_BLK, _D), jnp.float32), dims,
                                preferred_element_type=jnp.float32)

    @pl.when(i == pl.num_programs(0) - 1)
    def _():
        po_ref[...] = pooled[...]
        co_ref[...] = cnt[...]


def _tc_layer(epsp, x, agg, batch3, w, b):
    return pl.pallas_call(
        _layer_body,
        grid=(_N // _BLK,),
        in_specs=[
            pl.BlockSpec((1, 1), lambda i: (0, 0)),
            pl.BlockSpec((_BLK, _D), lambda i: (i, 0)),
            pl.BlockSpec((2, _BLK, _D), lambda i: (0, i, 0)),
            pl.BlockSpec((1, 1, _BLK), lambda i: (i, 0, 0)),
            pl.BlockSpec((_D, _D), lambda i: (0, 0)),
            pl.BlockSpec((1, _D), lambda i: (0, 0)),
        ],
        out_specs=[
            pl.BlockSpec((_BLK, _D), lambda i: (i, 0)),
            pl.BlockSpec((_G, _D), lambda i: (0, 0)),
            pl.BlockSpec((_G, _D), lambda i: (0, 0)),
        ],
        out_shape=[
            jax.ShapeDtypeStruct((_N, _D), jnp.float32),
            jax.ShapeDtypeStruct((_G, _D), jnp.float32),
            jax.ShapeDtypeStruct((_G, _D), jnp.float32),
        ],
        scratch_shapes=[
            pltpu.VMEM((_G, _D), jnp.float32),
            pltpu.VMEM((_G, _D), jnp.float32),
        ],
    )(epsp, x, agg, batch3, w, b)


def _final_body(p_ref, c_ref, w_ref, b_ref, o_ref):
    o_ref[...] = (jnp.dot(p_ref[...], w_ref[...],
                          preferred_element_type=jnp.float32)
                  + c_ref[...] * b_ref[...])


def _tc_final(pooled, cnt, w, b):
    return pl.pallas_call(
        _final_body,
        out_shape=jax.ShapeDtypeStruct((_G, _D), jnp.float32),
    )(pooled, cnt, w, b)


def kernel(x, edge_index, batch, eps0, W0, b0, eps1, W1, b1, eps2, W2, b2):
    src = edge_index[0].reshape(32, _NCH, _CH)
    dst = edge_index[1].reshape(32, _NCH, _CH)
    batch3 = batch.reshape(_N // _BLK, 1, _BLK)
    epss = jnp.stack([(1.0 + eps0).reshape(1, 1),
                      (1.0 + eps1).reshape(1, 1),
                      (1.0 + eps2).reshape(1, 1)])
    ws = jnp.stack([W0, W1, W2])
    bs = jnp.stack([b0.reshape(1, _D), b1.reshape(1, _D),
                    b2.reshape(1, _D)])

    def _scan_body(h, lw):
        eps_l, w_l, b_l = lw
        agg = _edge_agg(src, dst, h)
        h2, pooled, cnt = _tc_layer(eps_l, h, agg, batch3, w_l, b_l)
        return h2, (pooled, cnt)

    _, (pooleds, cnts) = lax.scan(_scan_body, x, (epss, ws, bs))
    return _tc_final(pooleds[-1], cnts[-1], W2, b2.reshape(1, _D))
